# Initial kernel scaffold; baseline (speedup 1.0000x reference)
import jax
import jax.numpy as jnp
from jax.experimental import pallas as pl


def kernel(anchor_set_id, dists_array, embeds, W, b):
    raise NotImplementedError("write your pallas kernel here")


# speedup vs baseline: 9.9160x; 9.9160x over previous
import jax
import jax.numpy as jnp
from jax import lax
from jax.experimental import pallas as pl
from jax.experimental.pallas import tpu as pltpu

_N = 10000
_A = 32
_E = 128
_P625 = 625
_TILE = 1000
_GRID = _N // _TILE


def _tc_body(anchor_ref, dt_ref, e_ref, w_ref, b_ref, out_ref, p_scr, h_scr):
    t = pl.program_id(0)

    @pl.when(t == 0)
    def _init():
        E = e_ref[...]
        W1 = w_ref[:, :_E]
        W2 = w_ref[:, _E:]
        B2 = e_ref[...].reshape(_P625, 16, _E).sum(axis=1)
        r_io = lax.broadcasted_iota(jnp.int32, (_P625, _P625), 0)
        j_io = lax.broadcasted_iota(jnp.int32, (_P625, _P625), 1)
        perm = (jnp.equal((2 * r_io) % _P625, j_io)
                | jnp.equal((2 * r_io + 1) % _P625, j_io)).astype(jnp.float32)
        S = jnp.dot(perm, B2, preferred_element_type=jnp.float32)
        h_scr[...] = lax.dot_general(
            S, W2, (((1,), (1,)), ((), ())),
            preferred_element_type=jnp.float32) * (1.0 / _A)
        rows = lax.broadcasted_iota(jnp.int32, (_A, _N), 1)
        oh = jnp.equal(anchor_ref[...], rows).astype(jnp.float32)
        A32 = jnp.dot(oh, E, preferred_element_type=jnp.float32)
        p_scr[...] = lax.dot_general(
            A32, W1, (((1,), (1,)), ((), ())),
            preferred_element_type=jnp.float32) * (1.0 / _A)

    m1 = jnp.dot(dt_ref[...], p_scr[...],
                 preferred_element_type=jnp.float32)
    n_io = lax.broadcasted_iota(jnp.int32, (_TILE, _P625), 0) + t * _TILE
    j2 = lax.broadcasted_iota(jnp.int32, (_TILE, _P625), 1)
    phi = jnp.equal(n_io % _P625, j2).astype(jnp.float32)
    m2 = jnp.dot(phi, h_scr[...], preferred_element_type=jnp.float32)
    out_ref[...] = m1 + m2 + b_ref[...]


def kernel(anchor_set_id, dists_array, embeds, W, b):
    anchor2d = anchor_set_id.reshape(_A, 1)
    dists_t = dists_array.T
    b2d = b.reshape(1, _E)
    return pl.pallas_call(
        _tc_body,
        grid=(_GRID,),
        in_specs=[
            pl.BlockSpec((_A, 1), lambda t: (0, 0)),
            pl.BlockSpec((_TILE, _A), lambda t: (t, 0)),
            pl.BlockSpec((_N, _E), lambda t: (0, 0)),
            pl.BlockSpec((_E, 2 * _E), lambda t: (0, 0)),
            pl.BlockSpec((1, _E), lambda t: (0, 0)),
        ],
        out_specs=pl.BlockSpec((_TILE, _E), lambda t: (t, 0)),
        out_shape=jax.ShapeDtypeStruct((_N, _E), jnp.float32),
        scratch_shapes=[
            pltpu.VMEM((_A, _E), jnp.float32),
            pltpu.VMEM((_P625, _E), jnp.float32),
        ],
    )(anchor2d, dists_t, embeds, W, b2d)
